# Initial kernel scaffold; baseline (speedup 1.0000x reference)
import jax
import jax.numpy as jnp
from jax.experimental import pallas as pl


def kernel(X, adj, mask, W1, b1, W2, b2, w_a):
    raise NotImplementedError("write your pallas kernel here")


# speedup vs baseline: 1.9265x; 1.9265x over previous
import functools
import math

import jax
import jax.numpy as jnp
from jax import lax
from jax.experimental import pallas as pl

B, N, D_IN, D_H = 8, 1000, 256, 256
PERCENT, EPS = 0.25, 1e-10
K = int(math.ceil(PERCENT * N))


def _body(x_ref, adj_ref, mrow_ref, mcol_ref, w1_ref, b1_ref, w2_ref, b2_ref,
          wa_ref, out_ref, h_ref, nadj_ref, nmask_ref):
    f32 = jnp.float32
    adj = adj_ref[0]
    x = x_ref[0]
    mrow = mrow_ref[0]
    mcol = mcol_ref[0]

    ax = jnp.dot(adj, x, preferred_element_type=f32)
    h1 = jnp.dot(ax, w1_ref[...], preferred_element_type=f32) + b1_ref[...]
    ah = jnp.dot(adj, h1, preferred_element_type=f32)
    h2 = jnp.dot(ah, w2_ref[...], preferred_element_type=f32) + b2_ref[...]
    hidden = mcol * h2

    msum = jnp.sum(mrow)
    out_ref[0] = jnp.sum(hidden, axis=0, keepdims=True) / (EPS + msum)

    logits_col = (jnp.dot(hidden, wa_ref[...], preferred_element_type=f32)
                  + (mcol - 1.0) * 1e10)
    lmax = jnp.max(logits_col)
    e_col = jnp.exp(logits_col - lmax)
    att_col = e_col / jnp.sum(e_col)
    att_row = jnp.transpose(att_col)

    z = att_col * hidden

    ii = lax.broadcasted_iota(jnp.int32, (N, N), 0)
    jj = lax.broadcasted_iota(jnp.int32, (N, N), 1)
    q = (att_col > att_row) | ((att_col == att_row) & (ii < jj))
    rank = jnp.sum(q.astype(jnp.int32), axis=0, keepdims=True)

    k_i = jnp.ceil(PERCENT * msum).astype(jnp.int32)
    r_iota = lax.broadcasted_iota(jnp.int32, (K, N), 0)
    p = jnp.where((rank == r_iota) & (r_iota < k_i), 1.0, 0.0)

    gathered = jnp.dot(p, adj, preferred_element_type=f32)
    colsum = jnp.sum(gathered, axis=0, keepdims=True)
    assign = gathered / (colsum + EPS)

    t1 = jnp.dot(assign, adj, preferred_element_type=f32)
    nadj_ref[0] = lax.dot_general(t1, assign, (((1,), (1,)), ((), ())),
                                  preferred_element_type=f32)
    h_ref[0] = jnp.dot(assign, z, preferred_element_type=f32)
    nmask_ref[0] = (lax.broadcasted_iota(jnp.int32, (1, K), 1)
                    < k_i).astype(f32)


def kernel(X, adj, mask, W1, b1, W2, b2, w_a):
    mrow = mask.reshape(B, 1, N)
    mcol = mask.reshape(B, N, 1)
    grid = (B,)
    full = lambda b: (0, 0)
    out, H, new_adj, new_mask = pl.pallas_call(
        _body,
        grid=grid,
        in_specs=[
            pl.BlockSpec((1, N, D_IN), lambda b: (b, 0, 0)),
            pl.BlockSpec((1, N, N), lambda b: (b, 0, 0)),
            pl.BlockSpec((1, 1, N), lambda b: (b, 0, 0)),
            pl.BlockSpec((1, N, 1), lambda b: (b, 0, 0)),
            pl.BlockSpec((D_IN, D_H), full),
            pl.BlockSpec((1, D_H), full),
            pl.BlockSpec((D_H, D_H), full),
            pl.BlockSpec((1, D_H), full),
            pl.BlockSpec((D_H, 1), full),
        ],
        out_specs=[
            pl.BlockSpec((1, 1, D_H), lambda b: (b, 0, 0)),
            pl.BlockSpec((1, K, D_H), lambda b: (b, 0, 0)),
            pl.BlockSpec((1, K, K), lambda b: (b, 0, 0)),
            pl.BlockSpec((1, 1, K), lambda b: (b, 0, 0)),
        ],
        out_shape=[
            jax.ShapeDtypeStruct((B, 1, D_H), jnp.float32),
            jax.ShapeDtypeStruct((B, K, D_H), jnp.float32),
            jax.ShapeDtypeStruct((B, K, K), jnp.float32),
            jax.ShapeDtypeStruct((B, 1, K), jnp.float32),
        ],
    )(X, adj, mrow, mcol, W1, b1.reshape(1, D_H), W2, b2.reshape(1, D_H),
      w_a.reshape(D_H, 1))
    return (out.reshape(B, D_H), H, new_adj, new_mask.reshape(B, K))


# speedup vs baseline: 2.2189x; 1.1518x over previous
import functools
import math

import jax
import jax.numpy as jnp
from jax import lax
from jax.experimental import pallas as pl

B, N, D_IN, D_H = 8, 1000, 256, 256
PERCENT, EPS = 0.25, 1e-10
K = int(math.ceil(PERCENT * N))


def _body(x_ref, adj_ref, mask_ref, w1_ref, b1_ref, w2_ref, b2_ref,
          wa_ref, out_ref, h_ref, nadj_ref, nmask_ref):
    f32 = jnp.float32
    b = pl.program_id(0)
    adj = adj_ref[0]
    x = x_ref[0]
    mrow = mask_ref[pl.ds(b, 1), :]
    mcol = jnp.transpose(mrow)
    b1 = b1_ref[...].reshape(1, D_H)
    b2 = b2_ref[...].reshape(1, D_H)

    ax = jnp.dot(adj, x, preferred_element_type=f32)
    h1 = jnp.dot(ax, w1_ref[...], preferred_element_type=f32) + b1
    ah = jnp.dot(adj, h1, preferred_element_type=f32)
    h2 = jnp.dot(ah, w2_ref[...], preferred_element_type=f32) + b2
    hidden = mcol * h2

    msum = jnp.sum(mrow)
    out_ref[pl.ds(b, 1), :] = jnp.sum(hidden, axis=0, keepdims=True) / (EPS + msum)

    logits_col = (jnp.dot(hidden, wa_ref[0], preferred_element_type=f32)
                  + (mcol - 1.0) * 1e10)
    lmax = jnp.max(logits_col)
    e_col = jnp.exp(logits_col - lmax)
    att_col = e_col / jnp.sum(e_col)
    att_row = jnp.transpose(att_col)

    z = att_col * hidden

    ii = lax.broadcasted_iota(jnp.int32, (N, N), 0)
    jj = lax.broadcasted_iota(jnp.int32, (N, N), 1)
    q = (att_col > att_row) | ((att_col == att_row) & (ii < jj))
    rank = jnp.sum(q.astype(jnp.int32), axis=0, keepdims=True)

    k_i = jnp.ceil(PERCENT * msum).astype(jnp.int32)
    r_iota = lax.broadcasted_iota(jnp.int32, (K, N), 0)
    p = jnp.where((rank == r_iota) & (r_iota < k_i), 1.0, 0.0)

    gathered = jnp.dot(p, adj, preferred_element_type=f32)
    colsum = jnp.sum(gathered, axis=0, keepdims=True)
    assign = gathered / (colsum + EPS)

    t1 = jnp.dot(assign, adj, preferred_element_type=f32)
    nadj_ref[0] = lax.dot_general(t1, assign, (((1,), (1,)), ((), ())),
                                  preferred_element_type=f32)
    h_ref[0] = jnp.dot(assign, z, preferred_element_type=f32)
    nmask_ref[pl.ds(b, 1), :] = (lax.broadcasted_iota(jnp.int32, (1, K), 1)
                                 < k_i).astype(f32)


def kernel(X, adj, mask, W1, b1, W2, b2, w_a):
    grid = (B,)
    full2 = lambda b: (0, 0)
    out, H, new_adj, new_mask = pl.pallas_call(
        _body,
        grid=grid,
        in_specs=[
            pl.BlockSpec((1, N, D_IN), lambda b: (b, 0, 0)),
            pl.BlockSpec((1, N, N), lambda b: (b, 0, 0)),
            pl.BlockSpec((B, N), full2),
            pl.BlockSpec((D_IN, D_H), full2),
            pl.BlockSpec((D_H,), lambda b: (0,)),
            pl.BlockSpec((D_H, D_H), full2),
            pl.BlockSpec((D_H,), lambda b: (0,)),
            pl.BlockSpec((1, D_H, 1), lambda b: (0, 0, 0)),
        ],
        out_specs=[
            pl.BlockSpec((B, D_H), full2),
            pl.BlockSpec((1, K, D_H), lambda b: (b, 0, 0)),
            pl.BlockSpec((1, K, K), lambda b: (b, 0, 0)),
            pl.BlockSpec((B, K), full2),
        ],
        out_shape=[
            jax.ShapeDtypeStruct((B, D_H), jnp.float32),
            jax.ShapeDtypeStruct((B, K, D_H), jnp.float32),
            jax.ShapeDtypeStruct((B, K, K), jnp.float32),
            jax.ShapeDtypeStruct((B, K), jnp.float32),
        ],
    )(X, adj, mask, W1, b1, W2, b2, w_a)
    return (out, H, new_adj, new_mask)


# speedup vs baseline: 2.5536x; 1.1508x over previous
import functools
import math

import jax
import jax.numpy as jnp
from jax import lax
from jax.experimental import pallas as pl

B, N, D_IN, D_H = 8, 1000, 256, 256
PERCENT, EPS = 0.25, 1e-10
K = int(math.ceil(PERCENT * N))


def _body(x_ref, adj_ref, mask_ref, w1_ref, b1_ref, w2_ref, b2_ref,
          wa_ref, out_ref, h_ref, nadj_ref, nmask_ref):
    f32 = jnp.float32
    b = pl.program_id(0)
    adj = adj_ref[0]
    x = x_ref[0]
    mrow = mask_ref[pl.ds(b, 1), :]
    mcol = jnp.transpose(mrow)
    b1 = b1_ref[...].reshape(1, D_H)
    b2 = b2_ref[...].reshape(1, D_H)
    wa_row = wa_ref[...].reshape(1, D_H)

    ax = jnp.dot(adj, x, preferred_element_type=f32)
    h1 = jnp.dot(ax, w1_ref[...], preferred_element_type=f32) + b1
    ah = jnp.dot(adj, h1, preferred_element_type=f32)
    h2 = jnp.dot(ah, w2_ref[...], preferred_element_type=f32) + b2
    hidden = mcol * h2

    msum = jnp.sum(mrow)
    out_ref[pl.ds(b, 1), :] = jnp.sum(hidden, axis=0, keepdims=True) / (EPS + msum)

    logits_col = (jnp.sum(hidden * wa_row, axis=1, keepdims=True)
                  + (mcol - 1.0) * 1e10)
    lmax = jnp.max(logits_col)
    e_col = jnp.exp(logits_col - lmax)
    att_col = e_col / jnp.sum(e_col)
    att_row = jnp.transpose(att_col)

    z = att_col * hidden

    ii = lax.broadcasted_iota(jnp.int32, (N, N), 0)
    jj = lax.broadcasted_iota(jnp.int32, (N, N), 1)
    q = (att_col > att_row) | ((att_col == att_row) & (ii < jj))
    rank = jnp.sum(q.astype(jnp.int32), axis=0, keepdims=True)

    k_i = jnp.ceil(PERCENT * msum).astype(jnp.int32)
    r_iota = lax.broadcasted_iota(jnp.int32, (K, N), 0)
    p = jnp.where((rank == r_iota) & (r_iota < k_i), 1.0, 0.0)

    gathered = jnp.dot(p, adj, preferred_element_type=f32)
    colsum = jnp.sum(gathered, axis=0, keepdims=True)
    assign = gathered / (colsum + EPS)

    t1 = jnp.dot(assign, adj, preferred_element_type=f32)
    nadj = lax.dot_general(t1, assign, (((1,), (1,)), ((), ())),
                           preferred_element_type=f32)
    nadj_ref[:, pl.ds(b, 1), :] = nadj.reshape(K, 1, K)
    h_out = jnp.dot(assign, z, preferred_element_type=f32)
    h_ref[:, pl.ds(b, 1), :] = h_out.reshape(K, 1, D_H)
    nmask_ref[pl.ds(b, 1), :] = (lax.broadcasted_iota(jnp.int32, (1, K), 1)
                                 < k_i).astype(f32)


def kernel(X, adj, mask, W1, b1, W2, b2, w_a):
    grid = (B,)
    full2 = lambda b: (0, 0)
    out, H, new_adj, new_mask = pl.pallas_call(
        _body,
        grid=grid,
        in_specs=[
            pl.BlockSpec((1, N, D_IN), lambda b: (b, 0, 0)),
            pl.BlockSpec((1, N, N), lambda b: (b, 0, 0)),
            pl.BlockSpec((B, N), full2),
            pl.BlockSpec((D_IN, D_H), full2),
            pl.BlockSpec((D_H,), lambda b: (0,)),
            pl.BlockSpec((D_H, D_H), full2),
            pl.BlockSpec((D_H,), lambda b: (0,)),
            pl.BlockSpec((D_H,), lambda b: (0,)),
        ],
        out_specs=[
            pl.BlockSpec((B, D_H), full2),
            pl.BlockSpec((K, B, D_H), lambda b: (0, 0, 0)),
            pl.BlockSpec((K, B, K), lambda b: (0, 0, 0)),
            pl.BlockSpec((B, K), full2),
        ],
        out_shape=[
            jax.ShapeDtypeStruct((B, D_H), jnp.float32),
            jax.ShapeDtypeStruct((K, B, D_H), jnp.float32),
            jax.ShapeDtypeStruct((K, B, K), jnp.float32),
            jax.ShapeDtypeStruct((B, K), jnp.float32),
        ],
    )(X, adj, mask, W1, b1, W2, b2, w_a.reshape(D_H))
    return (out, jnp.transpose(H, (1, 0, 2)), jnp.transpose(new_adj, (1, 0, 2)),
            new_mask)
